# baseline (speedup 1.0000x reference)
import functools

import jax
import jax.numpy as jnp
from jax import lax
from jax.experimental import pallas as pl
from jax.experimental.pallas import tpu as pltpu
from jax.experimental.pallas import tpu_sc as plsc

K_CODES = 8192
LATENT = 32
BETA = 0.25
TOK_BLK = 1024
CODE_BLK = 512
SEQ_BLK = 2048
N_SEQ = K_CODES // SEQ_BLK
N_SUB = SEQ_BLK // CODE_BLK
N_TOKENS = 8192


def _vq_body(z_ref, e_ref, sz_ref, se_ref, idx_ref, loss_ref):
    z = z_ref[...]
    e = e_ref[...]
    sz = sz_ref[...]

    run_val = jnp.full((TOK_BLK, 1), jnp.inf, dtype=jnp.float32)
    pick_val = jnp.zeros((TOK_BLK, 1), dtype=jnp.float32)
    run_idx = jnp.zeros((TOK_BLK, 1), dtype=jnp.int32)
    for b in range(N_SEQ):
        blk_val = jnp.full((TOK_BLK, 1), jnp.inf, dtype=jnp.float32)
        blk_idx = jnp.zeros((TOK_BLK, 1), dtype=jnp.int32)
        for j in range(N_SUB):
            c = b * SEQ_BLK + j * CODE_BLK
            eb = e[c:c + CODE_BLK, :]
            se = se_ref[0:1, c:c + CODE_BLK]
            mm = lax.dot_general(z, eb, (((1,), (1,)), ((), ())),
                                 preferred_element_type=jnp.float32)
            dis = (sz + se) - 2.0 * mm
            cmin = jnp.min(dis, axis=1, keepdims=True)
            cidx = jnp.argmin(dis, axis=1).astype(jnp.int32).reshape(
                TOK_BLK, 1) + c
            upd = cmin < blk_val
            blk_val = jnp.where(upd, cmin, blk_val)
            blk_idx = jnp.where(upd, cidx, blk_idx)
        upd = blk_val < run_val
        run_idx = jnp.where(upd, blk_idx, run_idx)
        pick_val = jnp.where(upd, blk_val, pick_val)
        run_val = jnp.where(
            upd, blk_val.astype(jnp.bfloat16).astype(jnp.float32), run_val)

    idx_ref[...] = run_idx
    loss_ref[...] = jnp.sum(pick_val).reshape(1, 1, 1)


def _loss_body(part_ref, loss_ref):
    m = jnp.sum(part_ref[...]) * (1.0 / (N_TOKENS * LATENT))
    loss_ref[...] = (m + BETA * m).reshape(1, 1)


def _gather_rows(table_pad, idx_flat):
    info = plsc.get_sparse_core_info()
    nw = info.num_cores * info.num_subcores
    rows_per_w = N_TOKENS // nw

    @functools.partial(
        pl.kernel,
        mesh=plsc.VectorSubcoreMesh(core_axis_name="c", subcore_axis_name="s"),
        out_type=jax.ShapeDtypeStruct((N_TOKENS, 128), jnp.float32),
        scratch_types=[
            pltpu.VMEM((128,), jnp.int32),
            pltpu.VMEM((128,), jnp.int32),
            pltpu.VMEM((128, 128), jnp.float32),
            pltpu.VMEM((128, 128), jnp.float32),
            pltpu.SemaphoreType.DMA,
        ],
    )
    def k(table_hbm, idx_hbm, out_hbm, idx_a, idx_b, rows_a, rows_b, sem):
        wid = lax.axis_index("s") * info.num_cores + lax.axis_index("c")
        base = wid * rows_per_w
        pltpu.sync_copy(idx_hbm.at[pl.ds(base, 128)], idx_a)
        pltpu.sync_copy(idx_hbm.at[pl.ds(base + 128, 128)], idx_b)
        cp_a = pltpu.async_copy(table_hbm.at[idx_a], rows_a, sem)
        cp_b = pltpu.async_copy(table_hbm.at[idx_b], rows_b, sem)
        cp_a.wait()
        cp_b.wait()
        pltpu.sync_copy(rows_a, out_hbm.at[pl.ds(base, 128)])
        pltpu.sync_copy(rows_b, out_hbm.at[pl.ds(base + 128, 128)])

    return k(table_pad, idx_flat)


def _vq(z_flat, embedding_weight, sz, se):
    grid = N_TOKENS // TOK_BLK
    idx2d, part = pl.pallas_call(
        _vq_body,
        grid=(grid,),
        in_specs=[
            pl.BlockSpec((TOK_BLK, LATENT), lambda i: (i, 0)),
            pl.BlockSpec((K_CODES, LATENT), lambda i: (0, 0)),
            pl.BlockSpec((TOK_BLK, 1), lambda i: (i, 0)),
            pl.BlockSpec((1, K_CODES), lambda i: (0, 0)),
        ],
        out_specs=[
            pl.BlockSpec((TOK_BLK, 1), lambda i: (i, 0)),
            pl.BlockSpec((1, 1, 1), lambda i: (i, 0, 0)),
        ],
        out_shape=[
            jax.ShapeDtypeStruct((N_TOKENS, 1), jnp.int32),
            jax.ShapeDtypeStruct((grid, 1, 1), jnp.float32),
        ],
        compiler_params=pltpu.CompilerParams(
            dimension_semantics=("parallel",)),
    )(z_flat, embedding_weight, sz, se)

    loss = pl.pallas_call(
        _loss_body,
        out_shape=jax.ShapeDtypeStruct((1, 1), jnp.float32),
    )(part)
    return idx2d, loss


def kernel(z, embedding_weight):
    zp = jnp.transpose(z, (0, 2, 3, 1))
    z_flat = zp.reshape(-1, LATENT)
    sz = jnp.sum(z_flat ** 2, axis=1, keepdims=True)
    se = jnp.sum(embedding_weight ** 2, axis=1).reshape(1, K_CODES)

    idx2d, loss = _vq(z_flat, embedding_weight, sz, se)
    idx = idx2d.reshape(-1)

    table_pad = jnp.pad(embedding_weight, ((0, 0), (0, 128 - LATENT)))
    zq_flat = _gather_rows(table_pad, idx)[:, :LATENT]
    z_q = jnp.transpose(zq_flat.reshape(zp.shape), (0, 3, 1, 2))
    return (z_q, idx, loss.reshape(()))


# speedup vs baseline: 1.5392x; 1.5392x over previous
import functools

import jax
import jax.numpy as jnp
from jax import lax
from jax.experimental import pallas as pl
from jax.experimental.pallas import tpu as pltpu
from jax.experimental.pallas import tpu_sc as plsc

K_CODES = 8192
LATENT = 32
BETA = 0.25
TOK_BLK = 2048
CODE_BLK = 512
SEQ_BLK = 2048
N_SEQ = K_CODES // SEQ_BLK
N_SUB = SEQ_BLK // CODE_BLK
N_TOKENS = 8192


def _vq_body(z_ref, e_ref, sz_ref, se_ref, idx_ref, loss_ref):
    z = z_ref[...]
    e = e_ref[...]
    sz = sz_ref[...]

    run_val = jnp.full((TOK_BLK, 1), jnp.inf, dtype=jnp.float32)
    pick_val = jnp.zeros((TOK_BLK, 1), dtype=jnp.float32)
    run_idx = jnp.zeros((TOK_BLK, 1), dtype=jnp.int32)
    for b in range(N_SEQ):
        blk_val = jnp.full((TOK_BLK, 1), jnp.inf, dtype=jnp.float32)
        blk_idx = jnp.zeros((TOK_BLK, 1), dtype=jnp.int32)
        for j in range(N_SUB):
            c = b * SEQ_BLK + j * CODE_BLK
            eb = e[c:c + CODE_BLK, :]
            se = se_ref[0:1, c:c + CODE_BLK]
            mm = lax.dot_general(z, eb, (((1,), (1,)), ((), ())),
                                 preferred_element_type=jnp.float32)
            dis = (sz + se) - 2.0 * mm
            cmin = jnp.min(dis, axis=1, keepdims=True)
            iota = lax.broadcasted_iota(jnp.int32, dis.shape, 1)
            cidx = jnp.min(jnp.where(dis == cmin, iota, K_CODES),
                           axis=1, keepdims=True) + c
            upd = cmin < blk_val
            blk_val = jnp.where(upd, cmin, blk_val)
            blk_idx = jnp.where(upd, cidx, blk_idx)
        upd = blk_val < run_val
        run_idx = jnp.where(upd, blk_idx, run_idx)
        pick_val = jnp.where(upd, blk_val, pick_val)
        run_val = jnp.where(
            upd, blk_val.astype(jnp.bfloat16).astype(jnp.float32), run_val)

    idx_ref[...] = run_idx
    loss_ref[...] = jnp.sum(pick_val).reshape(1, 1, 1)


def _loss_body(part_ref, loss_ref):
    m = jnp.sum(part_ref[...]) * (1.0 / (N_TOKENS * LATENT))
    loss_ref[...] = (m + BETA * m).reshape(1, 1)


def _gather_rows(table_pad, idx_flat):
    info = plsc.get_sparse_core_info()
    nw = info.num_cores * info.num_subcores
    rows_per_w = N_TOKENS // nw

    @functools.partial(
        pl.kernel,
        mesh=plsc.VectorSubcoreMesh(core_axis_name="c", subcore_axis_name="s"),
        out_type=jax.ShapeDtypeStruct((N_TOKENS, 128), jnp.float32),
        scratch_types=[
            pltpu.VMEM((128,), jnp.int32),
            pltpu.VMEM((128,), jnp.int32),
            pltpu.VMEM((128, 128), jnp.float32),
            pltpu.VMEM((128, 128), jnp.float32),
            pltpu.SemaphoreType.DMA,
        ],
    )
    def k(table_hbm, idx_hbm, out_hbm, idx_a, idx_b, rows_a, rows_b, sem):
        wid = lax.axis_index("s") * info.num_cores + lax.axis_index("c")
        base = wid * rows_per_w
        pltpu.sync_copy(idx_hbm.at[pl.ds(base, 128)], idx_a)
        pltpu.sync_copy(idx_hbm.at[pl.ds(base + 128, 128)], idx_b)
        cp_a = pltpu.async_copy(table_hbm.at[idx_a], rows_a, sem)
        cp_b = pltpu.async_copy(table_hbm.at[idx_b], rows_b, sem)
        cp_a.wait()
        cp_b.wait()
        pltpu.sync_copy(rows_a, out_hbm.at[pl.ds(base, 128)])
        pltpu.sync_copy(rows_b, out_hbm.at[pl.ds(base + 128, 128)])

    return k(table_pad, idx_flat)


def _vq(z_flat, embedding_weight, sz, se):
    grid = N_TOKENS // TOK_BLK
    idx2d, part = pl.pallas_call(
        _vq_body,
        grid=(grid,),
        in_specs=[
            pl.BlockSpec((TOK_BLK, LATENT), lambda i: (i, 0)),
            pl.BlockSpec((K_CODES, LATENT), lambda i: (0, 0)),
            pl.BlockSpec((TOK_BLK, 1), lambda i: (i, 0)),
            pl.BlockSpec((1, K_CODES), lambda i: (0, 0)),
        ],
        out_specs=[
            pl.BlockSpec((TOK_BLK, 1), lambda i: (i, 0)),
            pl.BlockSpec((1, 1, 1), lambda i: (i, 0, 0)),
        ],
        out_shape=[
            jax.ShapeDtypeStruct((N_TOKENS, 1), jnp.int32),
            jax.ShapeDtypeStruct((grid, 1, 1), jnp.float32),
        ],
        compiler_params=pltpu.CompilerParams(
            dimension_semantics=("parallel",)),
    )(z_flat, embedding_weight, sz, se)

    loss = pl.pallas_call(
        _loss_body,
        out_shape=jax.ShapeDtypeStruct((1, 1), jnp.float32),
    )(part)
    return idx2d, loss


def kernel(z, embedding_weight):
    zp = jnp.transpose(z, (0, 2, 3, 1))
    z_flat = zp.reshape(-1, LATENT)
    sz = jnp.sum(z_flat ** 2, axis=1, keepdims=True)
    se = jnp.sum(embedding_weight ** 2, axis=1).reshape(1, K_CODES)

    idx2d, loss = _vq(z_flat, embedding_weight, sz, se)
    idx = idx2d.reshape(-1)

    table_pad = jnp.pad(embedding_weight, ((0, 0), (0, 128 - LATENT)))
    zq_flat = _gather_rows(table_pad, idx)[:, :LATENT]
    z_q = jnp.transpose(zq_flat.reshape(zp.shape), (0, 3, 1, 2))
    return (z_q, idx, loss.reshape(()))
